# baseline (speedup 1.0000x reference)
import functools

import jax
import jax.numpy as jnp
import numpy as np
from jax.experimental import pallas as pl
from jax.experimental.pallas import tpu as pltpu

MAX_KPTS = 1024
NUM_PAIRS = 256
MAX_RADIUS = 16
NMS_RADIUS = 3
BLOCK_SIZE = 3
SINKHORN_ITERS = 20
EPSILON = 1.0
UNUSED_SCORE = 1.0
SCORE_THRESHOLD = 0.0

_rng = np.random.RandomState(42)
_PAIRS = jnp.asarray(
    _rng.randint(-MAX_RADIUS, MAX_RADIUS + 1, size=(NUM_PAIRS, 2, 2)).astype(np.float32))
_SOBEL_X = jnp.asarray([[-1., 0., 1.], [-2., 0., 2.], [-1., 0., 1.]], jnp.float32)
_SOBEL_Y = _SOBEL_X.T

N1 = MAX_KPTS + 1


def _conv2d(x, k):
    return jax.lax.conv_general_dilated(x, k[None, None], (1, 1), 'SAME')


def _shi_tomasi(img):
    ix = _conv2d(img, _SOBEL_X)
    iy = _conv2d(img, _SOBEL_Y)
    box = jnp.ones((BLOCK_SIZE, BLOCK_SIZE), jnp.float32) / float(BLOCK_SIZE * BLOCK_SIZE)
    ixx = _conv2d(ix * ix, box)
    iyy = _conv2d(iy * iy, box)
    ixy = _conv2d(ix * iy, box)
    tr = 0.5 * (ixx + iyy)
    rad = jnp.sqrt(jnp.maximum(0.25 * (ixx - iyy) ** 2 + ixy ** 2, 1e-12))
    return (tr - rad)[:, 0]


def _nms(scores):
    k = 2 * NMS_RADIUS + 1
    lm = jax.lax.reduce_window(scores, -jnp.inf, jax.lax.max, (1, k, k), (1, 1, 1),
                               [(0, 0), (NMS_RADIUS, NMS_RADIUS), (NMS_RADIUS, NMS_RADIUS)])
    return (scores >= lm - 1e-7).astype(jnp.float32)


def _select_topk(scores, nms_mask):
    b, h, w = scores.shape
    m = MAX_RADIUS
    yi = jnp.arange(h)
    xi = jnp.arange(w)
    yv = ((yi >= m) & (yi < h - m)).astype(jnp.float32)
    xv = ((xi >= m) & (xi < w - m)).astype(jnp.float32)
    border = yv[None, :, None] * xv[None, None, :]
    sm = scores * nms_mask * border
    sm = jnp.where(sm > SCORE_THRESHOLD, sm, jnp.zeros_like(sm))
    flat = sm.reshape(b, -1)
    ts, ti = jax.lax.top_k(flat, MAX_KPTS)
    y = (ti // w).astype(jnp.float32)
    x = (ti % w).astype(jnp.float32)
    kpts = jnp.stack([y, x], axis=-1)
    valid = (ts > 0).astype(jnp.float32)
    kpts = jnp.where(valid[..., None] > 0.5, kpts, -jnp.ones_like(kpts))
    return kpts, ts * valid


def _bad_desc(image, kpts):
    b, _, h, w = image.shape
    pos = kpts[:, :, None, None, :] + _PAIRS[None, None]
    y = jnp.clip(jnp.round(pos[..., 0]), 0, h - 1).astype(jnp.int32)
    x = jnp.clip(jnp.round(pos[..., 1]), 0, w - 1).astype(jnp.int32)
    img = image[:, 0]
    vals = jax.vmap(lambda im, yy, xx: im[yy, xx])(img, y, x)
    d = vals[..., 0] - vals[..., 1]
    return d / jnp.sqrt(jnp.sum(d * d, axis=-1, keepdims=True) + 1e-12)


_LOG_MU = -float(np.log(N1))


def _sinkhorn_body(d1_ref, d2_ref, out_ref, z_ref):
    d1 = d1_ref[0]
    d2 = d2_ref[0]
    g = jax.lax.dot_general(d1, d2, (((1,), (1,)), ((), ())),
                            preferred_element_type=jnp.float32)
    n1 = jnp.sum(d1 * d1, axis=1, keepdims=True)
    n2 = jnp.sum(d2 * d2, axis=1, keepdims=True)
    dist = jnp.sqrt(jnp.maximum(n1 + n2.T - 2.0 * g, 1e-12))
    z_ref[:, :] = jnp.full((N1, N1), UNUSED_SCORE / EPSILON, jnp.float32)
    z_ref[0:MAX_KPTS, 0:MAX_KPTS] = -dist / EPSILON

    z = z_ref[:, :]

    def body(_, uv):
        u, v = uv
        a = z + v
        m = jnp.max(a, axis=1, keepdims=True)
        u = _LOG_MU - (jnp.log(jnp.sum(jnp.exp(a - m), axis=1, keepdims=True)) + m)
        b = z + u
        m2 = jnp.max(b, axis=0, keepdims=True)
        v = _LOG_MU - (jnp.log(jnp.sum(jnp.exp(b - m2), axis=0, keepdims=True)) + m2)
        return u, v

    u0 = jnp.zeros((N1, 1), jnp.float32)
    v0 = jnp.zeros((1, N1), jnp.float32)
    u, v = jax.lax.fori_loop(0, SINKHORN_ITERS, body, (u0, v0))
    out_ref[0] = jnp.exp(z + u + v)


def _sinkhorn_pallas(d1, d2):
    b = d1.shape[0]
    return pl.pallas_call(
        _sinkhorn_body,
        grid=(b,),
        in_specs=[
            pl.BlockSpec((1, MAX_KPTS, NUM_PAIRS), lambda i: (i, 0, 0)),
            pl.BlockSpec((1, MAX_KPTS, NUM_PAIRS), lambda i: (i, 0, 0)),
        ],
        out_specs=pl.BlockSpec((1, N1, N1), lambda i: (i, 0, 0)),
        out_shape=jax.ShapeDtypeStruct((b, N1, N1), jnp.float32),
        scratch_shapes=[pltpu.VMEM((N1, N1), jnp.float32)],
    )(d1, d2)


def kernel(image1, image2):
    s1 = _shi_tomasi(image1)
    s2 = _shi_tomasi(image2)
    kpts1, _ = _select_topk(s1, _nms(s1))
    kpts2, _ = _select_topk(s2, _nms(s2))
    d1 = _bad_desc(image1, kpts1)
    d2 = _bad_desc(image2, kpts2)
    probs = _sinkhorn_pallas(d1, d2)
    return kpts1, kpts2, probs
